# baseline (speedup 1.0000x reference)
import functools

import jax
import jax.numpy as jnp
from jax import lax
from jax.experimental import pallas as pl
from jax.experimental.pallas import tpu as pltpu
from jax.experimental.pallas import tpu_sc as plsc

_C = 80
_NSUB = 16
_NCORE = 2
_L = 16
_DF = 40
_DZ = 40
_NP = 4


def _precompute(feat, w_fc, b_fc):
    n, d = feat.shape
    s = w_fc.shape[0]
    nb = 2000

    def body(f_ref, w_ref, b_ref, a_ref, bo_ref):
        f = f_ref[...]
        w = w_ref[0]
        contract = (((1,), (1,)), ((), ()))
        a_ref[0] = lax.dot_general(
            f, w[:, :d], contract, preferred_element_type=jnp.float32) + b_ref[0]
        bo_ref[0] = lax.dot_general(
            f, w[:, d:], contract, preferred_element_type=jnp.float32)

    return pl.pallas_call(
        body,
        grid=(s, n // nb),
        in_specs=[
            pl.BlockSpec((nb, d), lambda k, j: (j, 0)),
            pl.BlockSpec((1, d, 2 * d), lambda k, j: (k, 0, 0)),
            pl.BlockSpec((1, 1, d), lambda k, j: (k, 0, 0)),
        ],
        out_specs=[
            pl.BlockSpec((1, nb, d), lambda k, j: (k, j, 0)),
            pl.BlockSpec((1, nb, d), lambda k, j: (k, j, 0)),
        ],
        out_shape=[
            jax.ShapeDtypeStruct((s, n, d), jnp.float32),
            jax.ShapeDtypeStruct((s, n, d), jnp.float32),
        ],
    )(feat, w_fc, b_fc.reshape(s, 1, d))


def _make_sc(n, npad, e, s, d):
    ept = e // _NSUB
    nchunk = ept // _C
    rows = npad // _NSUB
    spc = s // _NCORE

    mesh = plsc.VectorSubcoreMesh(core_axis_name="c", subcore_axis_name="s",
                                  num_cores=_NCORE)

    @functools.partial(
        pl.kernel,
        out_type=jax.ShapeDtypeStruct((s * _NP * npad, _DZ), jnp.float32),
        mesh=mesh,
        compiler_params=pltpu.CompilerParams(use_tc_tiling_on_sc=False),
        scratch_types=[
            pltpu.VMEM((_C,), jnp.int32),
            pltpu.VMEM((_C,), jnp.int32),
            pltpu.VMEM((_C,), jnp.int32),
            pltpu.VMEM((_C,), jnp.int32),
            pltpu.VMEM((_C, d), jnp.float32),
            pltpu.VMEM((_C, d), jnp.float32),
            pltpu.VMEM((_C, d), jnp.float32),
            pltpu.VMEM((_C, d), jnp.float32),
            pltpu.VMEM((_C, _DF), jnp.float32),
            pltpu.VMEM((_C, _DF), jnp.float32),
            pltpu.VMEM((_C, _DZ), jnp.float32),
            pltpu.VMEM((d,), jnp.float32),
            pltpu.VMEM((e // _NSUB,), jnp.float32),
            pltpu.VMEM_SHARED((npad, _DZ), jnp.float32),
            pltpu.SemaphoreType.DMA,
            pltpu.SemaphoreType.DMA,
        ],
    )
    def sc_kernel(a_hbm, b_hbm, f0_hbm, f1_hbm, f2_hbm, f3_hbm, ei_hbm,
                  wo_hbm, z_hbm, i0a, i0b, i1a, i1b, a0, b0, a1, b1,
                  fv0, fv1, msg_v, wo_v, sval_v, z_sh, semA, semB):
        cid = lax.axis_index("c")
        sid = lax.axis_index("s")
        base = sid * rows
        lanes = lax.iota(jnp.int32, _L)
        dn = lax.GatherDimensionNumbers(
            offset_dims=(), collapsed_slice_dims=(0,), start_index_map=(0,))
        sbufs = ((i0a, i0b, a0, b0, semA), (i1a, i1b, a1, b1, semB))

        for k2 in range(spc):
            k = cid * spc + k2
            pltpu.sync_copy(wo_hbm.at[pl.ds(pl.multiple_of(k * d, 8), d)], wo_v)
            ebase = k * 4 * e + sid * ept

            def s_issue(c, buf):
                si, di, av, bv, sm = buf
                off = pl.multiple_of(ebase + c * _C, 8)
                pltpu.sync_copy(ei_hbm.at[pl.ds(off, _C)], si)
                pltpu.sync_copy(ei_hbm.at[pl.ds(off + e, _C)], di)
                pltpu.async_copy(a_hbm.at[si], av, sm)
                pltpu.async_copy(b_hbm.at[di], bv, sm)

            def s_compute(c, buf):
                si, di, av, bv, sm = buf
                pltpu.make_async_copy(a_hbm.at[si], av, sm).wait()
                pltpu.make_async_copy(b_hbm.at[di], bv, sm).wait()

                def group(g, _):
                    s_g = jnp.zeros((_L,), jnp.float32)
                    for ei in range(_L):
                        i = g * _L + ei
                        acc = jnp.zeros((_L,), jnp.float32)
                        for j in range(d // _L):
                            x = (av[i, pl.ds(j * _L, _L)]
                                 + bv[i, pl.ds(j * _L, _L)])
                            t = 2.0 / (1.0 + jnp.exp(-2.0 * x)) - 1.0
                            acc = acc + t * wo_v[pl.ds(j * _L, _L)]
                        for sh in (1, 2, 4, 8):
                            perm = jnp.bitwise_xor(lanes, sh)
                            acc = acc + lax.gather(
                                acc, perm[:, None], dn, slice_sizes=(1,),
                                mode=lax.GatherScatterMode.PROMISE_IN_BOUNDS)
                        s_g = jnp.where(lanes == ei, acc, s_g)
                    sval_v[pl.ds(c * _C + g * _L, _L)] = jnp.exp(s_g)
                    return 0
                lax.fori_loop(0, _C // _L, group, 0)

            assert nchunk % 2 == 0
            s_issue(0, sbufs[0])
            s_issue(1, sbufs[1])

            def spair(cc, _):
                c0 = 2 * cc
                s_compute(c0, sbufs[0])

                @pl.when(c0 + 2 < nchunk)
                def _():
                    s_issue(c0 + 2, sbufs[0])
                s_compute(c0 + 1, sbufs[1])

                @pl.when(c0 + 3 < nchunk)
                def _():
                    s_issue(c0 + 3, sbufs[1])
                return 0
            lax.fori_loop(0, nchunk // 2, spair, 0)

            for p, f_hbm in enumerate((f0_hbm, f1_hbm, f2_hbm, f3_hbm)):
                mbufs = ((i0a, i0b, fv0, semA), (i1a, i1b, fv1, semB))

                def zrow(i, _):
                    for j in (0, 16, 24):
                        msg_v[i, pl.ds(j, _L)] = jnp.zeros((_L,), jnp.float32)
                    return 0
                lax.fori_loop(0, _C, zrow, 0)
                for t in range(rows // _C):
                    pltpu.sync_copy(msg_v, z_sh.at[pl.ds(base + t * _C, _C)])
                rem = rows % _C
                if rem:
                    pltpu.sync_copy(
                        msg_v.at[pl.ds(0, rem)],
                        z_sh.at[pl.ds(base + (rows // _C) * _C, rem)])
                plsc.subcore_barrier()

                def m_issue(c, buf):
                    sp, dp, fv, sm = buf
                    off = pl.multiple_of(ebase + c * _C, 8)
                    pltpu.sync_copy(ei_hbm.at[pl.ds(off + 2 * e, _C)], sp)
                    pltpu.sync_copy(ei_hbm.at[pl.ds(off + 3 * e, _C)], dp)
                    pltpu.async_copy(f_hbm.at[sp], fv, sm)

                def m_compute(c, buf):
                    sp, dp, fv, sm = buf
                    pltpu.make_async_copy(f_hbm.at[sp], fv, sm).wait()

                    def mgroup(g, _):
                        sv16 = sval_v[pl.ds(c * _C + g * _L, _L)]
                        for ei in range(_L):
                            i = g * _L + ei
                            splat = jnp.full((_L, 1), ei, jnp.int32)
                            sv = lax.gather(
                                sv16, splat, dn, slice_sizes=(1,),
                                mode=lax.GatherScatterMode.PROMISE_IN_BOUNDS)
                            for j in (0, 16, 24):
                                msg_v[i, pl.ds(j, _L)] = (
                                    fv[i, pl.ds(j, _L)] * sv)
                        return 0
                    lax.fori_loop(0, _C // _L, mgroup, 0)

                    pltpu.sync_copy(msg_v, z_sh.at[dp], add=True)

                m_issue(0, mbufs[0])
                m_issue(1, mbufs[1])

                def mpair(cc, _):
                    c0 = 2 * cc
                    m_compute(c0, mbufs[0])

                    @pl.when(c0 + 2 < nchunk)
                    def _():
                        m_issue(c0 + 2, mbufs[0])
                    m_compute(c0 + 1, mbufs[1])

                    @pl.when(c0 + 3 < nchunk)
                    def _():
                        m_issue(c0 + 3, mbufs[1])
                    return 0
                lax.fori_loop(0, nchunk // 2, mpair, 0)
                plsc.subcore_barrier()

                pltpu.sync_copy(
                    z_sh.at[pl.ds(base, rows)],
                    z_hbm.at[pl.ds((k * _NP + p) * npad + base, rows)])
                plsc.subcore_barrier()

    return sc_kernel


def _finalize(z, n, npad, d, s):
    nb = 2048
    h = d // _NP

    def body(z_ref, o_ref):
        blk = z_ref[0]
        den = jnp.maximum(blk[0, :, h:h + 1], 1e-16)
        o_ref[...] = jnp.concatenate(
            [blk[p, :, :h] for p in range(_NP)], axis=1) / den

    return pl.pallas_call(
        body,
        grid=(s, (npad + nb - 1) // nb),
        in_specs=[pl.BlockSpec((1, _NP, nb, _DZ), lambda k, j: (k, 0, j, 0))],
        out_specs=pl.BlockSpec((nb, d), lambda k, j: (j, k)),
        out_shape=jax.ShapeDtypeStruct((n, s * d), jnp.float32),
    )(z.reshape(s, _NP, npad, _DZ))


def kernel(feat, edge_index, attn_fc_w, attn_fc_b, attn_out_w):
    n, d = feat.shape
    s, _, e = edge_index.shape
    h = d // 2

    a_t, b_t = _precompute(feat, attn_fc_w, attn_fc_b)
    ones = jnp.ones((n, 1), jnp.float32)
    h = d // _NP
    zpad = jnp.zeros((n, _DF - h - 1), jnp.float32)
    ftabs = [jnp.concatenate([feat[:, p * h:(p + 1) * h], ones, zpad], axis=1)
             for p in range(_NP)]
    src = edge_index[:, 0, :]
    dst = edge_index[:, 1, :]
    offs = (jnp.arange(s, dtype=jnp.int32) * n)[:, None]
    ei4 = jnp.stack([src + offs, dst + offs, src, dst], axis=1)

    npad = ((n + 127) // 128) * 128
    sc = _make_sc(n, npad, e, s, d)
    z = sc(a_t.reshape(s * n, d), b_t.reshape(s * n, d), *ftabs,
           ei4.reshape(-1), attn_out_w.reshape(-1))
    return _finalize(z, n, npad, d, s)


# speedup vs baseline: 1.3514x; 1.3514x over previous
import functools

import jax
import jax.numpy as jnp
from jax import lax
from jax.experimental import pallas as pl
from jax.experimental.pallas import tpu as pltpu
from jax.experimental.pallas import tpu_sc as plsc

_C = 160
_NSUB = 16
_NCORE = 2
_L = 16
_DF = 40
_DZ = 40
_NP = 4


def _precompute(feat, w_fc, b_fc):
    n, d = feat.shape
    s = w_fc.shape[0]
    nb = 2000

    def body(f_ref, w_ref, b_ref, a_ref, bo_ref):
        f = f_ref[...]
        w = w_ref[0]
        contract = (((1,), (1,)), ((), ()))
        a_ref[0] = lax.dot_general(
            f, w[:, :d], contract, preferred_element_type=jnp.float32) + b_ref[0]
        bo_ref[0] = lax.dot_general(
            f, w[:, d:], contract, preferred_element_type=jnp.float32)

    return pl.pallas_call(
        body,
        grid=(s, n // nb),
        in_specs=[
            pl.BlockSpec((nb, d), lambda k, j: (j, 0)),
            pl.BlockSpec((1, d, 2 * d), lambda k, j: (k, 0, 0)),
            pl.BlockSpec((1, 1, d), lambda k, j: (k, 0, 0)),
        ],
        out_specs=[
            pl.BlockSpec((1, nb, d), lambda k, j: (k, j, 0)),
            pl.BlockSpec((1, nb, d), lambda k, j: (k, j, 0)),
        ],
        out_shape=[
            jax.ShapeDtypeStruct((s, n, d), jnp.float32),
            jax.ShapeDtypeStruct((s, n, d), jnp.float32),
        ],
    )(feat, w_fc, b_fc.reshape(s, 1, d))


def _make_sc(n, npad, e, s, d):
    ept = e // _NSUB
    nchunk = ept // _C
    rows = npad // _NSUB
    spc = s // _NCORE

    mesh = plsc.VectorSubcoreMesh(core_axis_name="c", subcore_axis_name="s",
                                  num_cores=_NCORE)

    @functools.partial(
        pl.kernel,
        out_type=jax.ShapeDtypeStruct((s * _NP * npad, _DZ), jnp.float32),
        mesh=mesh,
        compiler_params=pltpu.CompilerParams(use_tc_tiling_on_sc=False),
        scratch_types=[
            pltpu.VMEM((_C,), jnp.int32),
            pltpu.VMEM((_C,), jnp.int32),
            pltpu.VMEM((_C,), jnp.int32),
            pltpu.VMEM((_C,), jnp.int32),
            pltpu.VMEM((_C, d), jnp.float32),
            pltpu.VMEM((_C, d), jnp.float32),
            pltpu.VMEM((_C, _DF), jnp.float32),
            pltpu.VMEM((_C, _DF), jnp.float32),
            pltpu.VMEM((_C, _DZ), jnp.float32),
            pltpu.VMEM((d,), jnp.float32),
            pltpu.VMEM((e // _NSUB,), jnp.float32),
            pltpu.VMEM_SHARED((npad, _DZ), jnp.float32),
            pltpu.SemaphoreType.DMA,
            pltpu.SemaphoreType.DMA,
        ],
    )
    def sc_kernel(a_hbm, b_hbm, f0_hbm, f1_hbm, f2_hbm, f3_hbm, ei_hbm,
                  wo_hbm, z_hbm, i0a, i0b, i1a, i1b, a0, b0,
                  fv0, fv1, msg_v, wo_v, sval_v, z_sh, semA, semB):
        cid = lax.axis_index("c")
        sid = lax.axis_index("s")
        base = sid * rows
        lanes = lax.iota(jnp.int32, _L)
        dn = lax.GatherDimensionNumbers(
            offset_dims=(), collapsed_slice_dims=(0,), start_index_map=(0,))
        sbuf = (i0a, i0b, a0, b0, semA)

        for k2 in range(spc):
            k = cid * spc + k2
            pltpu.sync_copy(wo_hbm.at[pl.ds(pl.multiple_of(k * d, 8), d)], wo_v)
            ebase = k * 4 * e + sid * ept

            def s_issue(c, buf):
                si, di, av, bv, sm = buf
                off = pl.multiple_of(ebase + c * _C, 8)
                pltpu.sync_copy(ei_hbm.at[pl.ds(off, _C)], si)
                pltpu.sync_copy(ei_hbm.at[pl.ds(off + e, _C)], di)
                pltpu.async_copy(a_hbm.at[si], av, sm)
                pltpu.async_copy(b_hbm.at[di], bv, sm)

            def s_compute(c, buf):
                si, di, av, bv, sm = buf
                pltpu.make_async_copy(a_hbm.at[si], av, sm).wait()
                pltpu.make_async_copy(b_hbm.at[di], bv, sm).wait()

                def group(g, _):
                    s_g = jnp.zeros((_L,), jnp.float32)
                    for ei in range(_L):
                        i = g * _L + ei
                        acc = jnp.zeros((_L,), jnp.float32)
                        for j in range(d // _L):
                            x = (av[i, pl.ds(j * _L, _L)]
                                 + bv[i, pl.ds(j * _L, _L)])
                            t = 2.0 / (1.0 + jnp.exp(-2.0 * x)) - 1.0
                            acc = acc + t * wo_v[pl.ds(j * _L, _L)]
                        for sh in (1, 2, 4, 8):
                            perm = jnp.bitwise_xor(lanes, sh)
                            acc = acc + lax.gather(
                                acc, perm[:, None], dn, slice_sizes=(1,),
                                mode=lax.GatherScatterMode.PROMISE_IN_BOUNDS)
                        s_g = jnp.where(lanes == ei, acc, s_g)
                    sval_v[pl.ds(c * _C + g * _L, _L)] = jnp.exp(s_g)
                    return 0
                lax.fori_loop(0, _C // _L, group, 0)

            def schunk(c, _):
                s_issue(c, sbuf)
                s_compute(c, sbuf)
                return 0
            lax.fori_loop(0, nchunk, schunk, 0)

            for p, f_hbm in enumerate((f0_hbm, f1_hbm, f2_hbm, f3_hbm)):
                mbufs = ((i0a, i0b, fv0, semA), (i1a, i1b, fv1, semB))

                def zrow(i, _):
                    for j in (0, 16, 24):
                        msg_v[i, pl.ds(j, _L)] = jnp.zeros((_L,), jnp.float32)
                    return 0
                lax.fori_loop(0, _C, zrow, 0)
                for t in range(rows // _C):
                    pltpu.sync_copy(msg_v, z_sh.at[pl.ds(base + t * _C, _C)])
                rem = rows % _C
                if rem:
                    pltpu.sync_copy(
                        msg_v.at[pl.ds(0, rem)],
                        z_sh.at[pl.ds(base + (rows // _C) * _C, rem)])
                plsc.subcore_barrier()

                def m_issue(c, buf):
                    sp, dp, fv, sm = buf
                    off = pl.multiple_of(ebase + c * _C, 8)
                    pltpu.sync_copy(ei_hbm.at[pl.ds(off + 2 * e, _C)], sp)
                    pltpu.sync_copy(ei_hbm.at[pl.ds(off + 3 * e, _C)], dp)
                    pltpu.async_copy(f_hbm.at[sp], fv, sm)

                def m_compute(c, buf):
                    sp, dp, fv, sm = buf
                    pltpu.make_async_copy(f_hbm.at[sp], fv, sm).wait()

                    def mgroup(g, _):
                        sv16 = sval_v[pl.ds(c * _C + g * _L, _L)]
                        for ei in range(_L):
                            i = g * _L + ei
                            splat = jnp.full((_L, 1), ei, jnp.int32)
                            sv = lax.gather(
                                sv16, splat, dn, slice_sizes=(1,),
                                mode=lax.GatherScatterMode.PROMISE_IN_BOUNDS)
                            for j in (0, 16, 24):
                                msg_v[i, pl.ds(j, _L)] = (
                                    fv[i, pl.ds(j, _L)] * sv)
                        return 0
                    lax.fori_loop(0, _C // _L, mgroup, 0)

                    pltpu.sync_copy(msg_v, z_sh.at[dp], add=True)

                assert nchunk % 2 == 1
                m_issue(0, mbufs[0])

                def mpair(cc, _):
                    c0 = 2 * cc
                    m_issue(c0 + 1, mbufs[1])
                    m_compute(c0, mbufs[0])
                    m_issue(c0 + 2, mbufs[0])
                    m_compute(c0 + 1, mbufs[1])
                    return 0
                lax.fori_loop(0, nchunk // 2, mpair, 0)
                m_compute(nchunk - 1, mbufs[0])
                plsc.subcore_barrier()

                pltpu.sync_copy(
                    z_sh.at[pl.ds(base, rows)],
                    z_hbm.at[pl.ds((k * _NP + p) * npad + base, rows)])
                plsc.subcore_barrier()

    return sc_kernel


def _finalize(z, n, npad, d, s):
    nb = 2048
    h = d // _NP

    def body(z_ref, o_ref):
        blk = z_ref[0]
        den = jnp.maximum(blk[0, :, h:h + 1], 1e-16)
        o_ref[...] = jnp.concatenate(
            [blk[p, :, :h] for p in range(_NP)], axis=1) / den

    return pl.pallas_call(
        body,
        grid=(s, (npad + nb - 1) // nb),
        in_specs=[pl.BlockSpec((1, _NP, nb, _DZ), lambda k, j: (k, 0, j, 0))],
        out_specs=pl.BlockSpec((nb, d), lambda k, j: (j, k)),
        out_shape=jax.ShapeDtypeStruct((n, s * d), jnp.float32),
    )(z.reshape(s, _NP, npad, _DZ))


def kernel(feat, edge_index, attn_fc_w, attn_fc_b, attn_out_w):
    n, d = feat.shape
    s, _, e = edge_index.shape
    h = d // 2

    a_t, b_t = _precompute(feat, attn_fc_w, attn_fc_b)
    ones = jnp.ones((n, 1), jnp.float32)
    h = d // _NP
    zpad = jnp.zeros((n, _DF - h - 1), jnp.float32)
    ftabs = [jnp.concatenate([feat[:, p * h:(p + 1) * h], ones, zpad], axis=1)
             for p in range(_NP)]
    src = edge_index[:, 0, :]
    dst = edge_index[:, 1, :]
    offs = (jnp.arange(s, dtype=jnp.int32) * n)[:, None]
    ei4 = jnp.stack([src + offs, dst + offs, src, dst], axis=1)

    npad = ((n + 127) // 128) * 128
    sc = _make_sc(n, npad, e, s, d)
    z = sc(a_t.reshape(s * n, d), b_t.reshape(s * n, d), *ftabs,
           ei4.reshape(-1), attn_out_w.reshape(-1))
    return _finalize(z, n, npad, d, s)


# speedup vs baseline: 1.3642x; 1.0095x over previous
import functools

import jax
import jax.numpy as jnp
from jax import lax
from jax.experimental import pallas as pl
from jax.experimental.pallas import tpu as pltpu
from jax.experimental.pallas import tpu_sc as plsc

_C = 160
_NSUB = 16
_NCORE = 2
_L = 16
_DF = 40
_DZ = 40
_NP = 4


def _precompute(feat, w_fc, b_fc):
    n, d = feat.shape
    s = w_fc.shape[0]
    nb = 2000

    def body(f_ref, w_ref, b_ref, a_ref, bo_ref):
        f = f_ref[...]
        w = w_ref[0]
        contract = (((1,), (1,)), ((), ()))
        a_ref[0] = lax.dot_general(
            f, w[:, :d], contract, preferred_element_type=jnp.float32) + b_ref[0]
        bo_ref[0] = lax.dot_general(
            f, w[:, d:], contract, preferred_element_type=jnp.float32)

    return pl.pallas_call(
        body,
        grid=(s, n // nb),
        in_specs=[
            pl.BlockSpec((nb, d), lambda k, j: (j, 0)),
            pl.BlockSpec((1, d, 2 * d), lambda k, j: (k, 0, 0)),
            pl.BlockSpec((1, 1, d), lambda k, j: (k, 0, 0)),
        ],
        out_specs=[
            pl.BlockSpec((1, nb, d), lambda k, j: (k, j, 0)),
            pl.BlockSpec((1, nb, d), lambda k, j: (k, j, 0)),
        ],
        out_shape=[
            jax.ShapeDtypeStruct((s, n, d), jnp.float32),
            jax.ShapeDtypeStruct((s, n, d), jnp.float32),
        ],
    )(feat, w_fc, b_fc.reshape(s, 1, d))


def _make_sc(n, npad, e, s, d):
    ept = e // _NSUB
    nchunk = ept // _C
    rows = npad // _NSUB
    spc = s // _NCORE

    mesh = plsc.VectorSubcoreMesh(core_axis_name="c", subcore_axis_name="s",
                                  num_cores=_NCORE)

    @functools.partial(
        pl.kernel,
        out_type=jax.ShapeDtypeStruct((s * _NP * npad, _DZ), jnp.float32),
        mesh=mesh,
        compiler_params=pltpu.CompilerParams(use_tc_tiling_on_sc=False),
        scratch_types=[
            pltpu.VMEM((_C,), jnp.int32),
            pltpu.VMEM((_C,), jnp.int32),
            pltpu.VMEM((_C,), jnp.int32),
            pltpu.VMEM((_C,), jnp.int32),
            pltpu.VMEM((_C, d), jnp.float32),
            pltpu.VMEM((_C, d), jnp.float32),
            pltpu.VMEM((_C, _DF), jnp.float32),
            pltpu.VMEM((_C, _DF), jnp.float32),
            pltpu.VMEM((_C, _DZ), jnp.float32),
            pltpu.VMEM((d,), jnp.float32),
            pltpu.VMEM((e // _NSUB,), jnp.float32),
            pltpu.VMEM_SHARED((npad, _DZ), jnp.float32),
            pltpu.SemaphoreType.DMA,
            pltpu.SemaphoreType.DMA,
        ],
    )
    def sc_kernel(a_hbm, b_hbm, f0_hbm, f1_hbm, f2_hbm, f3_hbm, ei_hbm,
                  wo_hbm, z_hbm, i0a, i0b, i1a, i1b, a0, b0,
                  fv0, fv1, msg_v, wo_v, sval_v, z_sh, semA, semB):
        cid = lax.axis_index("c")
        sid = lax.axis_index("s")
        base = sid * rows
        lanes = lax.iota(jnp.int32, _L)
        dn = lax.GatherDimensionNumbers(
            offset_dims=(), collapsed_slice_dims=(0,), start_index_map=(0,))
        sbuf = (i0a, i0b, a0, b0, semA)

        for k2 in range(spc):
            k = cid * spc + k2
            pltpu.sync_copy(wo_hbm.at[pl.ds(pl.multiple_of(k * d, 8), d)], wo_v)
            ebase = k * 4 * e + sid * ept

            def s_issue(c, buf):
                si, di, av, bv, sm = buf
                off = pl.multiple_of(ebase + c * _C, 8)
                pltpu.sync_copy(ei_hbm.at[pl.ds(off, _C)], si)
                pltpu.sync_copy(ei_hbm.at[pl.ds(off + e, _C)], di)
                pltpu.async_copy(a_hbm.at[si], av, sm)
                pltpu.async_copy(b_hbm.at[di], bv, sm)

            def s_compute(c, buf):
                si, di, av, bv, sm = buf
                pltpu.make_async_copy(a_hbm.at[si], av, sm).wait()
                pltpu.make_async_copy(b_hbm.at[di], bv, sm).wait()

                def group(g, _):
                    accs = [jnp.zeros((_L,), jnp.float32) for _ in range(_L)]
                    for j in range(d // _L):
                        wj = wo_v[pl.ds(j * _L, _L)]
                        for ei in range(_L):
                            i = g * _L + ei
                            x = (av[i, pl.ds(j * _L, _L)]
                                 + bv[i, pl.ds(j * _L, _L)])
                            t = 2.0 / (1.0 + jnp.exp(-2.0 * x)) - 1.0
                            accs[ei] = accs[ei] + t * wj
                    s_g = jnp.zeros((_L,), jnp.float32)
                    for ei in range(_L):
                        acc = accs[ei]
                        for sh in (1, 2, 4, 8):
                            perm = jnp.bitwise_xor(lanes, sh)
                            acc = acc + lax.gather(
                                acc, perm[:, None], dn, slice_sizes=(1,),
                                mode=lax.GatherScatterMode.PROMISE_IN_BOUNDS)
                        s_g = jnp.where(lanes == ei, acc, s_g)
                    sval_v[pl.ds(c * _C + g * _L, _L)] = jnp.exp(s_g)
                    return 0
                lax.fori_loop(0, _C // _L, group, 0)

            def schunk(c, _):
                s_issue(c, sbuf)
                s_compute(c, sbuf)
                return 0
            lax.fori_loop(0, nchunk, schunk, 0)

            for p, f_hbm in enumerate((f0_hbm, f1_hbm, f2_hbm, f3_hbm)):
                mbufs = ((i0a, i0b, fv0, semA), (i1a, i1b, fv1, semB))

                def zrow(i, _):
                    for j in (0, 16, 24):
                        msg_v[i, pl.ds(j, _L)] = jnp.zeros((_L,), jnp.float32)
                    return 0
                lax.fori_loop(0, _C, zrow, 0)
                for t in range(rows // _C):
                    pltpu.sync_copy(msg_v, z_sh.at[pl.ds(base + t * _C, _C)])
                rem = rows % _C
                if rem:
                    pltpu.sync_copy(
                        msg_v.at[pl.ds(0, rem)],
                        z_sh.at[pl.ds(base + (rows // _C) * _C, rem)])
                plsc.subcore_barrier()

                def m_issue(c, buf):
                    sp, dp, fv, sm = buf
                    off = pl.multiple_of(ebase + c * _C, 8)
                    pltpu.sync_copy(ei_hbm.at[pl.ds(off + 2 * e, _C)], sp)
                    pltpu.sync_copy(ei_hbm.at[pl.ds(off + 3 * e, _C)], dp)
                    pltpu.async_copy(f_hbm.at[sp], fv, sm)

                def m_compute(c, buf):
                    sp, dp, fv, sm = buf
                    pltpu.make_async_copy(f_hbm.at[sp], fv, sm).wait()

                    def mgroup(g, _):
                        sv16 = sval_v[pl.ds(c * _C + g * _L, _L)]
                        for ei in range(_L):
                            i = g * _L + ei
                            splat = jnp.full((_L, 1), ei, jnp.int32)
                            sv = lax.gather(
                                sv16, splat, dn, slice_sizes=(1,),
                                mode=lax.GatherScatterMode.PROMISE_IN_BOUNDS)
                            for j in (0, 16, 24):
                                msg_v[i, pl.ds(j, _L)] = (
                                    fv[i, pl.ds(j, _L)] * sv)
                        return 0
                    lax.fori_loop(0, _C // _L, mgroup, 0)

                    pltpu.sync_copy(msg_v, z_sh.at[dp], add=True)

                assert nchunk % 2 == 1
                m_issue(0, mbufs[0])

                def mpair(cc, _):
                    c0 = 2 * cc
                    m_issue(c0 + 1, mbufs[1])
                    m_compute(c0, mbufs[0])
                    m_issue(c0 + 2, mbufs[0])
                    m_compute(c0 + 1, mbufs[1])
                    return 0
                lax.fori_loop(0, nchunk // 2, mpair, 0)
                m_compute(nchunk - 1, mbufs[0])
                plsc.subcore_barrier()

                pltpu.sync_copy(
                    z_sh.at[pl.ds(base, rows)],
                    z_hbm.at[pl.ds((k * _NP + p) * npad + base, rows)])
                plsc.subcore_barrier()

    return sc_kernel


def _finalize(z, n, npad, d, s):
    nb = 2048
    h = d // _NP

    def body(z_ref, o_ref):
        blk = z_ref[0]
        den = jnp.maximum(blk[0, :, h:h + 1], 1e-16)
        o_ref[...] = jnp.concatenate(
            [blk[p, :, :h] for p in range(_NP)], axis=1) / den

    return pl.pallas_call(
        body,
        grid=(s, (npad + nb - 1) // nb),
        in_specs=[pl.BlockSpec((1, _NP, nb, _DZ), lambda k, j: (k, 0, j, 0))],
        out_specs=pl.BlockSpec((nb, d), lambda k, j: (j, k)),
        out_shape=jax.ShapeDtypeStruct((n, s * d), jnp.float32),
    )(z.reshape(s, _NP, npad, _DZ))


def kernel(feat, edge_index, attn_fc_w, attn_fc_b, attn_out_w):
    n, d = feat.shape
    s, _, e = edge_index.shape
    h = d // 2

    a_t, b_t = _precompute(feat, attn_fc_w, attn_fc_b)
    ones = jnp.ones((n, 1), jnp.float32)
    h = d // _NP
    zpad = jnp.zeros((n, _DF - h - 1), jnp.float32)
    ftabs = [jnp.concatenate([feat[:, p * h:(p + 1) * h], ones, zpad], axis=1)
             for p in range(_NP)]
    src = edge_index[:, 0, :]
    dst = edge_index[:, 1, :]
    offs = (jnp.arange(s, dtype=jnp.int32) * n)[:, None]
    ei4 = jnp.stack([src + offs, dst + offs, src, dst], axis=1)

    npad = ((n + 127) // 128) * 128
    sc = _make_sc(n, npad, e, s, d)
    z = sc(a_t.reshape(s * n, d), b_t.reshape(s * n, d), *ftabs,
           ei4.reshape(-1), attn_out_w.reshape(-1))
    return _finalize(z, n, npad, d, s)
